# baseline (speedup 1.0000x reference)
import jax
import jax.numpy as jnp
from jax import lax
from jax.experimental import pallas as pl
from jax.experimental.pallas import tpu as pltpu
from jax.experimental.pallas import tpu_sc as plsc

NC = 2
NS = 16
L = 16
NW = NC * NS

DIM = 64
BLK = 128
NBUF = 4
LOOKAHEAD = 2


def _pack_body(wt_hbm, tail_hbm, packed_hbm, tbufs, obufs, gsems, osems):
    wid = lax.axis_index("s") * NC + lax.axis_index("c")
    vocab = wt_hbm.shape[1]
    n_full = vocab // BLK
    per_w = (n_full + NW - 1) // NW
    lane = lax.iota(jnp.int32, L)

    def fire_in(i, b):
        blk = i * NW + wid
        pltpu.async_copy(
            wt_hbm.at[:, pl.ds(blk * BLK, BLK)],
            tbufs[b].at[:, pl.ds(0, BLK)],
            gsems[b],
        )

    def wait_in(b):
        pltpu.make_async_copy(
            wt_hbm.at[:, pl.ds(0, BLK)], tbufs[b].at[:, pl.ds(0, BLK)], gsems[b]
        ).wait()

    def fire_out(i, b):
        blk = i * NW + wid
        pltpu.async_copy(
            obufs[b], packed_hbm.at[pl.ds(blk * BLK, BLK)], osems[b]
        )

    def wait_out(b):
        pltpu.make_async_copy(
            obufs[b], packed_hbm.at[pl.ds(0, BLK)], osems[b]
        ).wait()

    def transpose_rows(tbuf, obuf, nrows):
        @plsc.parallel_loop(0, nrows, unroll=2)
        def _(r):
            rvec = jnp.full((L,), 0, jnp.int32) + r
            for j in range(4):
                v = plsc.load_gather(tbuf, [lane + j * L, rvec])
                obuf[r, pl.ds(j * L, L)] = v

    def guarded(i, fn):
        @pl.when(i * NW + wid < n_full)
        def _():
            fn()

    for i in range(LOOKAHEAD):
        guarded(i, lambda i=i: fire_in(i, i % NBUF))

    def group_body(g, _):
        for b in range(NBUF):
            i = g * NBUF + b
            guarded(i, lambda b=b: wait_in(b))

            @pl.when((i >= NBUF) & (i * NW + wid < n_full))
            def _():
                wait_out(b)

            guarded(i, lambda b=b: transpose_rows(tbufs[b], obufs[b], BLK))
            guarded(i, lambda i=i, b=b: fire_out(i, b))
            guarded(
                i + LOOKAHEAD,
                lambda i=i, b=b: fire_in(i + LOOKAHEAD, (b + LOOKAHEAD) % NBUF),
            )
        return 0

    n_groups = (per_w + NBUF - 1) // NBUF
    lax.fori_loop(0, n_groups, group_body, 0)
    for k in range(NBUF):
        wait_out(k)

    tail = vocab - n_full * BLK
    if tail:

        @pl.when(wid == NW - 1)
        def _():
            pltpu.async_copy(
                tail_hbm.at[pl.ds(0, tail), :], obufs[0].at[pl.ds(0, tail), :],
                gsems[0],
            ).wait()
            pltpu.async_copy(
                obufs[0].at[pl.ds(0, tail), :],
                packed_hbm.at[pl.ds(n_full * BLK, tail)],
                osems[0],
            ).wait()


def _lookup_body(xt_hbm, packed_hbm, out_hbm, idx_v, gbufs, obufs, gsems, osems):
    wid = lax.axis_index("s") * NC + lax.axis_index("c")
    fields = xt_hbm.shape[0]
    batches = xt_hbm.shape[1]
    blocks_per_w = batches // (BLK * NW)
    n_chunks = blocks_per_w * fields
    base_blk = wid * blocks_per_w

    pltpu.sync_copy(xt_hbm.at[:, pl.ds(base_blk * BLK, blocks_per_w * BLK)], idx_v)

    lane = lax.iota(jnp.int32, L)
    chi_sel = [lax.div(lane + j * L, 8) for j in range(4)]
    clo_sel = lax.rem(lane, 8)

    def fire_gather(c, b):
        f = lax.rem(c, fields)
        bl = lax.div(c, fields)
        pltpu.async_copy(
            packed_hbm.at[idx_v.at[f, pl.ds(bl * BLK, BLK)]], gbufs[b], gsems[b]
        )

    def wait_gather(b):
        pltpu.make_async_copy(
            packed_hbm.at[idx_v.at[0, pl.ds(0, BLK)]], gbufs[b], gsems[b]
        ).wait()

    def fire_out(c, b):
        f = lax.rem(c, fields)
        bl = lax.div(c, fields)
        pltpu.async_copy(
            obufs[b].at[:, :, pl.ds(0, BLK)],
            out_hbm.at[f, :, base_blk + bl, :, :],
            osems[b],
        )

    def wait_out(b):
        pltpu.make_async_copy(
            obufs[b].at[:, :, pl.ds(0, BLK)], out_hbm.at[0, :, 0, :, :], osems[b]
        ).wait()

    def compute(b):
        gbuf, obuf = gbufs[b], obufs[b]

        @plsc.parallel_loop(0, BLK, unroll=2)
        def _(r):
            bvec = jnp.full((L,), 0, jnp.int32) + r
            for j in range(4):
                v = gbuf[r, pl.ds(j * L, L)]
                s = v / (1.0 + jnp.exp(-v))
                plsc.store_scatter(obuf, [chi_sel[j], clo_sel, bvec], s)

    for c in range(LOOKAHEAD):
        fire_gather(c, c % NBUF)

    def group_body(g, _):
        for b in range(NBUF):
            c = g * NBUF + b
            wait_gather(b)

            @pl.when(c >= NBUF)
            def _():
                wait_out(b)

            compute(b)
            fire_out(c, b)

            @pl.when(c + LOOKAHEAD < n_chunks)
            def _():
                fire_gather(c + LOOKAHEAD, (b + LOOKAHEAD) % NBUF)

        return 0

    lax.fori_loop(0, n_chunks // NBUF, group_body, 0)
    for k in range(NBUF):
        wait_out((n_chunks - NBUF + k) % NBUF)


@jax.jit
def kernel(x, emb_weight):
    batch, fields = x.shape
    vocab, dim = emb_weight.shape
    assert batch % (NW * BLK) == 0 and dim == DIM
    xt = x.T.astype(jnp.int32)
    wt = emb_weight.T

    mesh = plsc.VectorSubcoreMesh(
        core_axis_name="c", subcore_axis_name="s", num_cores=NC, num_subcores=NS
    )

    pack = pl.kernel(
        _pack_body,
        out_type=jax.ShapeDtypeStruct((vocab, 2 * dim), jnp.float32),
        mesh=mesh,
        scratch_types=[
            [pltpu.VMEM((dim, BLK + 1), jnp.float32) for _ in range(NBUF)],
            [pltpu.VMEM((BLK, 2 * dim), jnp.float32) for _ in range(NBUF)],
            [pltpu.SemaphoreType.DMA for _ in range(NBUF)],
            [pltpu.SemaphoreType.DMA for _ in range(NBUF)],
        ],
        compiler_params=pltpu.CompilerParams(needs_layout_passes=False),
    )
    tail_n = vocab % BLK
    tail_p = jnp.pad(
        emb_weight[vocab - tail_n :], ((0, BLK - tail_n), (0, 2 * dim - dim))
    )
    packed = pack(wt, tail_p)

    run = pl.kernel(
        _lookup_body,
        out_type=jax.ShapeDtypeStruct(
            (fields, dim // 8, batch // BLK, 8, BLK), jnp.float32
        ),
        mesh=mesh,
        scratch_types=[
            pltpu.VMEM((fields, batch // NW), jnp.int32),
            [pltpu.VMEM((BLK, 2 * dim), jnp.float32) for _ in range(NBUF)],
            [pltpu.VMEM((dim // 8, 8, BLK + 1), jnp.float32) for _ in range(NBUF)],
            [pltpu.SemaphoreType.DMA for _ in range(NBUF)],
            [pltpu.SemaphoreType.DMA for _ in range(NBUF)],
        ],
        compiler_params=pltpu.CompilerParams(
            use_tc_tiling_on_sc=False, needs_layout_passes=False
        ),
    )
    e = run(xt, packed)
    return e.transpose(2, 4, 0, 1, 3).reshape(batch, fields, dim)


# speedup vs baseline: 1.3055x; 1.3055x over previous
import jax
import jax.numpy as jnp
from jax import lax
from jax.experimental import pallas as pl
from jax.experimental.pallas import tpu as pltpu
from jax.experimental.pallas import tpu_sc as plsc

NC = 2
NS = 16
L = 16
NW = NC * NS

DIM = 64
BLK = 128
NBUF = 4
LOOKAHEAD = 2


def _sc_body(xt_hbm, table_hbm, out_hbm, idx_v, gbufs, obufs, gsems, osems):
    wid = lax.axis_index("s") * NC + lax.axis_index("c")
    fields = xt_hbm.shape[0]
    batches = xt_hbm.shape[1]
    blocks_per_w = batches // (BLK * NW)
    n_chunks = blocks_per_w * fields
    base_blk = wid * blocks_per_w

    pltpu.sync_copy(xt_hbm.at[:, pl.ds(base_blk * BLK, blocks_per_w * BLK)], idx_v)

    lane = lax.iota(jnp.int32, L)
    chi_sel = [lax.div(lane + j * L, 8) for j in range(4)]
    clo_sel = lax.rem(lane, 8)

    def fire_gather(c, b):
        f = lax.rem(c, fields)
        bl = lax.div(c, fields)
        pltpu.async_copy(
            table_hbm.at[idx_v.at[f, pl.ds(bl * BLK, BLK)]], gbufs[b], gsems[b]
        )

    def wait_gather(b):
        pltpu.make_async_copy(
            table_hbm.at[idx_v.at[0, pl.ds(0, BLK)]], gbufs[b], gsems[b]
        ).wait()

    def fire_out(c, b):
        f = lax.rem(c, fields)
        bl = lax.div(c, fields)
        pltpu.async_copy(
            obufs[b].at[:, :, pl.ds(0, BLK)],
            out_hbm.at[f, :, base_blk + bl, :, :],
            osems[b],
        )

    def wait_out(b):
        pltpu.make_async_copy(
            obufs[b].at[:, :, pl.ds(0, BLK)], out_hbm.at[0, :, 0, :, :], osems[b]
        ).wait()

    def compute(b):
        gbuf, obuf = gbufs[b], obufs[b]

        @plsc.parallel_loop(0, BLK, unroll=2)
        def _(r):
            bvec = jnp.full((L,), r, jnp.int32)
            for j in range(4):
                v = gbuf[r, pl.ds(j * L, L)]
                s = v / (1.0 + jnp.exp(-v))
                plsc.store_scatter(obuf, [chi_sel[j], clo_sel, bvec], s)

    for c in range(LOOKAHEAD):
        fire_gather(c, c % NBUF)

    def group_body(g, _):
        for b in range(NBUF):
            c = g * NBUF + b
            wait_gather(b)

            @pl.when(c >= NBUF)
            def _():
                wait_out(b)

            compute(b)
            fire_out(c, b)

            @pl.when(c + LOOKAHEAD < n_chunks)
            def _():
                fire_gather(c + LOOKAHEAD, (b + LOOKAHEAD) % NBUF)

        return 0

    lax.fori_loop(0, n_chunks // NBUF, group_body, 0)

    for k in range(NBUF):
        wait_out((n_chunks - NBUF + k) % NBUF)


@jax.jit
def kernel(x, emb_weight):
    batch, fields = x.shape
    dim = emb_weight.shape[1]
    assert batch % (NW * BLK) == 0 and dim == DIM
    xt = x.T.astype(jnp.int32)

    mesh = plsc.VectorSubcoreMesh(
        core_axis_name="c", subcore_axis_name="s", num_cores=NC, num_subcores=NS
    )
    run = pl.kernel(
        _sc_body,
        out_type=jax.ShapeDtypeStruct(
            (fields, dim // 8, batch // BLK, 8, BLK), jnp.float32
        ),
        mesh=mesh,
        scratch_types=[
            pltpu.VMEM((fields, batch // NW), jnp.int32),
            [pltpu.VMEM((BLK, dim), jnp.float32) for _ in range(NBUF)],
            [pltpu.VMEM((dim // 8, 8, BLK + 1), jnp.float32) for _ in range(NBUF)],
            [pltpu.SemaphoreType.DMA for _ in range(NBUF)],
            [pltpu.SemaphoreType.DMA for _ in range(NBUF)],
        ],
        compiler_params=pltpu.CompilerParams(
            use_tc_tiling_on_sc=False, needs_layout_passes=False
        ),
    )
    e = run(xt, emb_weight)
    return e.transpose(2, 4, 0, 1, 3).reshape(batch, fields, dim)
